# Initial kernel scaffold; baseline (speedup 1.0000x reference)
import jax
import jax.numpy as jnp
from jax.experimental import pallas as pl


def kernel(x, h, batch_idx, lig_flag, gen_flag, Wq, Wk, Wv, Uq, Uk, Uv):
    raise NotImplementedError("write your pallas kernel here")


# speedup vs baseline: 9.2088x; 9.2088x over previous
import functools
import math

import jax
import jax.numpy as jnp
from jax import lax
from jax.experimental import pallas as pl
from jax.experimental.pallas import tpu as pltpu
from jax.experimental.pallas import tpu_sc as plsc

N = 10000
K = 16
HD = 128
NH = 16
DH = HD // NH
NG = 20
RMAX = 10.0
L = 2
NB = 4

NPAD = 10240
EPAD = NPAD * K
BIG = 1e10
CONSUMED = 3e38
IBIG = 2**30


_KR = 256
_KC = 256
_NRB = NPAD // _KR
_NCB = NPAD // _KC


def _knn_body(xa_ref, xb_ref, x2r_ref, x2c_ref, lo_ref, hi_ref, out_ref,
              bd, bi):
    i = pl.program_id(0)
    j = pl.program_id(1)

    @pl.when(j == 0)
    def _init():
        bd[...] = jnp.full((_KR, K), CONSUMED, jnp.float32)
        bi[...] = jnp.zeros((_KR, K), jnp.int32)

    lo = lo_ref[...]
    hi = hi_ref[...]
    lo_min = jnp.min(lo)
    hi_max = jnp.max(hi)
    active = jnp.logical_and(j * _KC < hi_max, (j + 1) * _KC > lo_min)

    @pl.when(active)
    def _merge():
        xa = xa_ref[...]
        xb = xb_ref[...]
        m = lax.dot_general(xa, xb, (((1,), (1,)), ((), ())),
                            preferred_element_type=jnp.float32,
                            precision=lax.Precision.DEFAULT)
        d2 = (x2r_ref[...] + x2c_ref[0:1, :]) - 2.0 * m
        col = j * _KC + lax.broadcasted_iota(jnp.int32, (_KR, _KC), 1)
        row = i * _KR + lax.broadcasted_iota(jnp.int32, (_KR, _KC), 0)
        valid = (col >= lo) & (col < hi) & (col != row)
        d2 = jnp.where(valid, d2, BIG)
        buf = jnp.concatenate([bd[...], d2], axis=1)
        ibuf = jnp.concatenate([bi[...], col], axis=1)
        nd, ni = [], []
        for _ in range(K):
            m = jnp.min(buf, axis=1, keepdims=True)
            hit = buf == m
            im = jnp.min(jnp.where(hit, ibuf, IBIG), axis=1, keepdims=True)
            nd.append(m)
            ni.append(im)
            buf = jnp.where(hit & (ibuf == im), CONSUMED, buf)
        bd[...] = jnp.concatenate(nd, axis=1)
        bi[...] = jnp.concatenate(ni, axis=1)

    @pl.when(j == _NCB - 1)
    def _emit():
        out_ref[...] = bi[...]


def _knn(xa, xb, x2col, x2row, lo, hi):
    return pl.pallas_call(
        _knn_body,
        grid=(_NRB, _NCB),
        in_specs=[
            pl.BlockSpec((_KR, 8), lambda i, j: (i, 0)),
            pl.BlockSpec((_KC, 8), lambda i, j: (j, 0)),
            pl.BlockSpec((_KR, 1), lambda i, j: (i, 0)),
            pl.BlockSpec((8, _KC), lambda i, j: (0, j)),
            pl.BlockSpec((_KR, 1), lambda i, j: (i, 0)),
            pl.BlockSpec((_KR, 1), lambda i, j: (i, 0)),
        ],
        out_specs=pl.BlockSpec((_KR, K), lambda i, j: (i, 0)),
        out_shape=jax.ShapeDtypeStruct((NPAD, K), jnp.int32),
        scratch_shapes=[
            pltpu.VMEM((_KR, K), jnp.float32),
            pltpu.VMEM((_KR, K), jnp.int32),
        ],
        compiler_params=pltpu.CompilerParams(
            dimension_semantics=("arbitrary", "arbitrary")),
    )(xa, xb, x2col, x2row, lo, hi)


_SC_NC = 2
_SC_NS = 16
_NW = _SC_NC * _SC_NS


def _sc_gather(table, idx, chunk):
    B = idx.shape[0]
    D = table.shape[1]
    bpw = B // _NW
    nch = bpw // chunk
    mesh = plsc.VectorSubcoreMesh(core_axis_name="c", subcore_axis_name="s",
                                  num_cores=_SC_NC, num_subcores=_SC_NS)

    @functools.partial(
        pl.kernel,
        out_type=jax.ShapeDtypeStruct((B, D), jnp.float32),
        mesh=mesh,
        scratch_types=[
            pltpu.VMEM((chunk,), jnp.int32),
            pltpu.VMEM((chunk, D), jnp.float32),
            pltpu.SemaphoreType.DMA,
        ],
        compiler_params=pltpu.CompilerParams(use_tc_tiling_on_sc=False),
    )
    def k(table_hbm, idx_hbm, out_hbm, idx_v, rows_v, sem):
        wid = lax.axis_index("s") * _SC_NC + lax.axis_index("c")
        base = wid * bpw
        for c in range(nch):
            off = base + c * chunk
            pltpu.sync_copy(idx_hbm.at[pl.ds(off, chunk)], idx_v)
            pltpu.async_copy(table_hbm.at[idx_v], rows_v, sem).wait()
            pltpu.sync_copy(rows_v, out_hbm.at[pl.ds(off, chunk)])

    return k(table, idx)


_SR = 128
_SRK = _SR * K
_NSB = NPAD // _SR

_SIG = RMAX / NG
_MU_STEP = RMAX / (NG - 1)


def _edge_geom(xs, xd_exp):
    lane = lax.broadcasted_iota(jnp.int32, (_SRK, 16), 1)
    rel = jnp.where(lane < 3, xd_exp - xs, 0.0)
    dist = jnp.sqrt(jnp.sum(rel * rel, axis=1, keepdims=True) + 1e-12)
    mu = lax.broadcasted_iota(jnp.int32, (_SRK, NG), 1).astype(
        jnp.float32) * _MU_STEP
    feat = jnp.exp(-((dist - mu) ** 2) / (2.0 * _SIG * _SIG))
    return rel, dist, feat


def _expand(v):
    d = v.shape[1]
    return jnp.broadcast_to(v.reshape(_SR, 1, d), (_SR, K, d)).reshape(_SRK, d)


def _edge_type_weights(lig_s, lig_d):
    s1 = (lig_s > 0.5).astype(jnp.float32)
    d1 = (lig_d > 0.5).astype(jnp.float32)
    return (s1 * d1, s1 * (1.0 - d1), (1.0 - s1) * d1, (1.0 - s1) * (1.0 - d1))


def _mm(a, b):
    return lax.dot_general(a, b, (((1,), (0,)), ((), ())),
                           preferred_element_type=jnp.float32,
                           precision=lax.Precision.HIGHEST)


def _proj(feat, w, wr_ref, We, hsrc, Wh, dout):
    acc = _mm(hsrc, Wh)
    for t in range(4):
        part = _mm(feat, wr_ref[pl.ds(t * NG, NG), :])
        acc = acc + w[t] * (part + We[t, :].reshape(1, dout))
    return acc


def _head_sum_mat():
    r = lax.broadcasted_iota(jnp.int32, (HD, NH), 0)
    c = lax.broadcasted_iota(jnp.int32, (HD, NH), 1)
    return (r // DH == c).astype(jnp.float32)


def _head_exp_mat():
    r = lax.broadcasted_iota(jnp.int32, (NH, HD), 0)
    c = lax.broadcasted_iota(jnp.int32, (NH, HD), 1)
    return (c // DH == r).astype(jnp.float32)


def _soft_over_k(sc):
    s3 = sc.reshape(_SR, K, NH)
    m = s3[:, 0, :]
    for t in range(1, K):
        m = jnp.maximum(m, s3[:, t, :])
    m = jnp.where(jnp.isfinite(m), m, 0.0)
    a3 = jnp.exp(s3 - m.reshape(_SR, 1, NH))
    s = a3[:, 0, :]
    for t in range(1, K):
        s = s + a3[:, t, :]
    al3 = a3 / (s.reshape(_SR, 1, NH) + 1e-16)
    return al3.reshape(_SRK, NH)


def _ew_from0(x0s, x0d_exp):
    _, _, feat0 = _edge_geom(x0s, x0d_exp)
    mean0 = jnp.mean(feat0, axis=1, keepdims=True)
    return jax.nn.sigmoid(mean0)


def _stage1_body(h_ref, hs_ref, x0s_ref, xcs_ref, x0d_ref, xcd_ref,
                 wq_ref, wkr_ref, wke_ref, wkh_ref,
                 wvr_ref, wve_ref, wvh_ref, out_ref):
    h = h_ref[...]
    hs = hs_ref[...]
    x0s = x0s_ref[...]
    xcs = xcs_ref[...]
    x0d = x0d_ref[...]
    xcd = xcd_ref[...]

    x0d_exp = _expand(x0d)
    xcd_exp = _expand(xcd)
    rel, dist, feat = _edge_geom(xcs, xcd_exp)
    e_w = _ew_from0(x0s, x0d_exp)
    w = _edge_type_weights(x0s[:, 3:4], x0d_exp[:, 3:4])

    q = _mm(h, wq_ref[...])
    q_exp = _expand(q)
    k_ = _proj(feat, w, wkr_ref, wke_ref[...], hs, wkh_ref[...], HD)
    v_ = _proj(feat, w, wvr_ref, wve_ref[...], hs, wvh_ref[...], HD)

    S = _head_sum_mat()
    sc = _mm(q_exp * k_, S) * (1.0 / math.sqrt(DH))
    al = _soft_over_k(sc)
    al_exp = _mm(al, _head_exp_mat())
    msg = al_exp * v_ * e_w
    m3 = msg.reshape(_SR, K, HD)
    agg = m3[:, 0, :]
    for t in range(1, K):
        agg = agg + m3[:, t, :]
    z = h + agg
    mean = jnp.mean(z, axis=1, keepdims=True)
    var = jnp.mean((z - mean) ** 2, axis=1, keepdims=True)
    out_ref[...] = (z - mean) / jnp.sqrt(var + 1e-5)


def _stage2_body(h_ref, hs_ref, x0s_ref, xcs_ref, x0d_ref, xcd_ref,
                 uq_ref, ukr_ref, uke_ref, ukh_ref,
                 uvr_ref, uve_ref, uvh_ref, out_ref):
    h = h_ref[...]
    hs = hs_ref[...]
    x0s = x0s_ref[...]
    xcs = xcs_ref[...]
    x0d = x0d_ref[...]
    xcd = xcd_ref[...]

    x0d_exp = _expand(x0d)
    xcd_exp = _expand(xcd)
    rel, dist, feat = _edge_geom(xcs, xcd_exp)
    e_w = _ew_from0(x0s, x0d_exp)
    w = _edge_type_weights(x0s[:, 3:4], x0d_exp[:, 3:4])

    q = _mm(h, uq_ref[...])
    q_exp = _expand(q)
    k_ = _proj(feat, w, ukr_ref, uke_ref[...], hs, ukh_ref[...], HD)
    v_ = _proj(feat, w, uvr_ref, uve_ref[...], hs, uvh_ref[...], NH)

    S = _head_sum_mat()
    sc = _mm(q_exp * k_, S) * (1.0 / math.sqrt(DH))
    al = _soft_over_k(sc)
    coef = jnp.mean(al * v_, axis=1, keepdims=True) * e_w
    dvec = coef * rel / (dist + 1.0)
    d3 = dvec.reshape(_SR, K, 16)
    delta = d3[:, 0, :]
    for t in range(1, K):
        delta = delta + d3[:, t, :]
    gen = x0d[:, 4:5]
    out_ref[...] = xcd + delta * gen


def _stage_call(body, h, hsrc, x0s, xcs, x0d, xcd, wq, wr, we, wh,
                vr, ve, vh, out_dim):
    nblk = pl.BlockSpec((_SR, HD), lambda i: (i, 0))
    eblk16 = pl.BlockSpec((_SRK, 16), lambda i: (i, 0))
    nblk16 = pl.BlockSpec((_SR, 16), lambda i: (i, 0))

    def full(a):
        return pl.BlockSpec(a.shape, lambda i: tuple(0 for _ in a.shape))

    return pl.pallas_call(
        body,
        grid=(_NSB,),
        in_specs=[
            nblk,
            pl.BlockSpec((_SRK, HD), lambda i: (i, 0)),
            eblk16, eblk16, nblk16, nblk16,
            full(wq), full(wr), full(we), full(wh),
            full(vr), full(ve), full(vh),
        ],
        out_specs=pl.BlockSpec((_SR, out_dim), lambda i: (i, 0)),
        out_shape=jax.ShapeDtypeStruct((NPAD, out_dim), jnp.float32),
        compiler_params=pltpu.CompilerParams(
            dimension_semantics=("arbitrary",)),
    )(h, hsrc, x0s, xcs, x0d, xcd, wq, wr, we, wh, vr, ve, vh)


def _split_w(W):
    dout = W.shape[1]
    Wr = W[: 4 * NG]
    We = jnp.concatenate([W[4 * NG: 4 * NG + 4],
                          jnp.zeros((4, dout), jnp.float32)], axis=0)
    Wh = W[4 * NG + 4:]
    return Wr, We, Wh


def kernel(x, h, batch_idx, lig_flag, gen_flag, Wq, Wk, Wv, Uq, Uk, Uv):
    f32 = jnp.float32
    xp = jnp.zeros((NPAD, 3), f32).at[:N].set(x)
    hp = jnp.zeros((NPAD, HD), f32).at[:N].set(h)
    x2 = jnp.sum(x * x, axis=-1)
    x2p = jnp.zeros((NPAD,), f32).at[:N].set(x2)

    starts = jnp.searchsorted(batch_idx, jnp.arange(NB), side="left")
    ends = jnp.searchsorted(batch_idx, jnp.arange(NB), side="right")
    lo = jnp.zeros((NPAD,), jnp.int32).at[:N].set(
        starts[batch_idx].astype(jnp.int32))
    hi = jnp.zeros((NPAD,), jnp.int32).at[:N].set(
        ends[batch_idx].astype(jnp.int32))

    xp8 = jnp.concatenate([xp, jnp.zeros((NPAD, 5), f32)], axis=1)
    x2row = jnp.broadcast_to(x2p[None, :], (8, NPAD))

    idx = _knn(xp8, xp8, x2p[:, None], x2row,
               lo[:, None], hi[:, None])
    src = idx.reshape(-1)

    ligp = jnp.zeros((NPAD,), f32).at[:N].set(lig_flag.astype(f32))
    genp = jnp.zeros((NPAD,), f32).at[:N].set(gen_flag.astype(f32))

    def node16(xcur):
        return jnp.concatenate(
            [xcur, ligp[:, None], genp[:, None], jnp.zeros((NPAD, 11), f32)],
            axis=1)

    x0tab = node16(xp)
    x0src = _sc_gather(x0tab, src, 512)

    wq_l = [Wq[l] for l in range(L)]
    uq_l = [Uq[l] for l in range(L)]
    wk_l = [_split_w(Wk[l]) for l in range(L)]
    wv_l = [_split_w(Wv[l]) for l in range(L)]
    uk_l = [_split_w(Uk[l]) for l in range(L)]
    uv_l = [_split_w(Uv[l]) for l in range(L)]

    hcur = hp
    xcur_tab = x0tab
    xcur_src = x0src
    for l in range(L):
        hsrc = _sc_gather(hcur, src, 512)
        wr, we, wh = wk_l[l]
        vr, ve, vh = wv_l[l]
        hcur = _stage_call(_stage1_body, hcur, hsrc, x0src, xcur_src,
                           x0tab, xcur_tab, wq_l[l], wr, we, wh, vr, ve, vh,
                           HD)
        hsrc2 = _sc_gather(hcur, src, 512)
        wr, we, wh = uk_l[l]
        vr, ve, vh = uv_l[l]
        xcur_tab = _stage_call(_stage2_body, hcur, hsrc2, x0src, xcur_src,
                               x0tab, xcur_tab, uq_l[l], wr, we, wh,
                               vr, ve, vh, 16)
        if l + 1 < L:
            xcur_src = _sc_gather(xcur_tab, src, 512)

    x_out = xcur_tab[:N, :3]
    h_out = hcur[:N]
    return (x_out, h_out)


# speedup vs baseline: 21.3503x; 2.3185x over previous
import functools
import math

import jax
import jax.numpy as jnp
from jax import lax
from jax.experimental import pallas as pl
from jax.experimental.pallas import tpu as pltpu
from jax.experimental.pallas import tpu_sc as plsc

N = 10000
K = 16
HD = 128
NH = 16
DH = HD // NH
NG = 20
RMAX = 10.0
L = 2
NB = 4

NPAD = 10240
EPAD = NPAD * K
BIG = 1e10
CONSUMED = 3e38
IBIG = 2**30


_KR = 256
_KC = 1024
_NRB = NPAD // _KR
_NCB = NPAD // _KC


def _knn_body(xa_ref, xb_ref, x2r_ref, x2c_ref, lo_ref, hi_ref, out_ref,
              bd, bi):
    i = pl.program_id(0)
    j = pl.program_id(1)

    @pl.when(j == 0)
    def _init():
        bd[...] = jnp.full((_KR, K), CONSUMED, jnp.float32)
        bi[...] = jnp.zeros((_KR, K), jnp.int32)

    lo = lo_ref[...]
    hi = hi_ref[...]
    lo_min = jnp.min(lo)
    hi_max = jnp.max(hi)
    active = jnp.logical_and(j * _KC < hi_max, (j + 1) * _KC > lo_min)

    @pl.when(active)
    def _merge():
        xa = xa_ref[...]
        xb = xb_ref[...]
        m = lax.dot_general(xa, xb, (((1,), (1,)), ((), ())),
                            preferred_element_type=jnp.float32,
                            precision=lax.Precision.DEFAULT)
        d2 = (x2r_ref[...] + x2c_ref[0:1, :]) - 2.0 * m
        col = j * _KC + lax.broadcasted_iota(jnp.int32, (_KR, _KC), 1)
        row = i * _KR + lax.broadcasted_iota(jnp.int32, (_KR, _KC), 0)
        valid = (col >= lo) & (col < hi) & (col != row)
        d2 = jnp.where(valid, d2, BIG)
        buf = jnp.concatenate([bd[...], d2], axis=1)
        ibuf = jnp.concatenate([bi[...], col], axis=1)
        nd, ni = [], []
        for _ in range(K):
            m = jnp.min(buf, axis=1, keepdims=True)
            hit = buf == m
            im = jnp.min(jnp.where(hit, ibuf, IBIG), axis=1, keepdims=True)
            nd.append(m)
            ni.append(im)
            buf = jnp.where(hit & (ibuf == im), CONSUMED, buf)
        bd[...] = jnp.concatenate(nd, axis=1)
        bi[...] = jnp.concatenate(ni, axis=1)

    @pl.when(j == _NCB - 1)
    def _emit():
        out_ref[...] = bi[...]


def _knn(xa, xb, x2col, x2row, lo, hi):
    return pl.pallas_call(
        _knn_body,
        grid=(_NRB, _NCB),
        in_specs=[
            pl.BlockSpec((_KR, 8), lambda i, j: (i, 0)),
            pl.BlockSpec((_KC, 8), lambda i, j: (j, 0)),
            pl.BlockSpec((_KR, 1), lambda i, j: (i, 0)),
            pl.BlockSpec((8, _KC), lambda i, j: (0, j)),
            pl.BlockSpec((_KR, 1), lambda i, j: (i, 0)),
            pl.BlockSpec((_KR, 1), lambda i, j: (i, 0)),
        ],
        out_specs=pl.BlockSpec((_KR, K), lambda i, j: (i, 0)),
        out_shape=jax.ShapeDtypeStruct((NPAD, K), jnp.int32),
        scratch_shapes=[
            pltpu.VMEM((_KR, K), jnp.float32),
            pltpu.VMEM((_KR, K), jnp.int32),
        ],
        compiler_params=pltpu.CompilerParams(
            dimension_semantics=("arbitrary", "arbitrary")),
    )(xa, xb, x2col, x2row, lo, hi)


_SC_NC = 2
_SC_NS = 16
_NW = _SC_NC * _SC_NS


def _sc_gather(table, idx, chunk):
    B = idx.shape[0]
    D = table.shape[1]
    bpw = B // _NW
    nch = bpw // chunk
    mesh = plsc.VectorSubcoreMesh(core_axis_name="c", subcore_axis_name="s",
                                  num_cores=_SC_NC, num_subcores=_SC_NS)

    @functools.partial(
        pl.kernel,
        out_type=jax.ShapeDtypeStruct((B, D), jnp.float32),
        mesh=mesh,
        scratch_types=[
            pltpu.VMEM((chunk,), jnp.int32),
            pltpu.VMEM((chunk, D), jnp.float32),
            pltpu.SemaphoreType.DMA,
        ],
        compiler_params=pltpu.CompilerParams(use_tc_tiling_on_sc=False),
    )
    def k(table_hbm, idx_hbm, out_hbm, idx_v, rows_v, sem):
        wid = lax.axis_index("s") * _SC_NC + lax.axis_index("c")
        base = wid * bpw
        for c in range(nch):
            off = base + c * chunk
            pltpu.sync_copy(idx_hbm.at[pl.ds(off, chunk)], idx_v)
            pltpu.async_copy(table_hbm.at[idx_v], rows_v, sem).wait()
            pltpu.sync_copy(rows_v, out_hbm.at[pl.ds(off, chunk)])

    return k(table, idx)


_SR = 128
_SRK = _SR * K
_NSB = NPAD // _SR

_SIG = RMAX / NG
_MU_STEP = RMAX / (NG - 1)


def _edge_geom(xs, xd_exp):
    lane = lax.broadcasted_iota(jnp.int32, (_SRK, 16), 1)
    rel = jnp.where(lane < 3, xd_exp - xs, 0.0)
    dist = jnp.sqrt(jnp.sum(rel * rel, axis=1, keepdims=True) + 1e-12)
    mu = lax.broadcasted_iota(jnp.int32, (_SRK, NG), 1).astype(
        jnp.float32) * _MU_STEP
    feat = jnp.exp(-((dist - mu) ** 2) / (2.0 * _SIG * _SIG))
    return rel, dist, feat


def _expand(v):
    d = v.shape[1]
    return jnp.broadcast_to(v.reshape(_SR, 1, d), (_SR, K, d)).reshape(_SRK, d)


def _edge_type_weights(lig_s, lig_d):
    s1 = (lig_s > 0.5).astype(jnp.float32)
    d1 = (lig_d > 0.5).astype(jnp.float32)
    return (s1 * d1, s1 * (1.0 - d1), (1.0 - s1) * d1, (1.0 - s1) * (1.0 - d1))


def _mm(a, b):
    return lax.dot_general(a, b, (((1,), (0,)), ((), ())),
                           preferred_element_type=jnp.float32,
                           precision=lax.Precision.DEFAULT)


def _proj(feat, w, wr_ref, We, hsrc, Wh, dout):
    acc = _mm(hsrc, Wh)
    for t in range(4):
        part = _mm(feat, wr_ref[pl.ds(t * NG, NG), :])
        acc = acc + w[t] * (part + We[t, :].reshape(1, dout))
    return acc


def _head_sum_mat():
    r = lax.broadcasted_iota(jnp.int32, (HD, NH), 0)
    c = lax.broadcasted_iota(jnp.int32, (HD, NH), 1)
    return (r // DH == c).astype(jnp.float32)


def _head_exp_mat():
    r = lax.broadcasted_iota(jnp.int32, (NH, HD), 0)
    c = lax.broadcasted_iota(jnp.int32, (NH, HD), 1)
    return (c // DH == r).astype(jnp.float32)


def _soft_over_k(sc):
    s3 = sc.reshape(_SR, K, NH)
    m = jnp.max(s3, axis=1)
    m = jnp.where(jnp.isfinite(m), m, 0.0)
    a3 = jnp.exp(s3 - m.reshape(_SR, 1, NH))
    s = jnp.sum(a3, axis=1)
    al3 = a3 / (s.reshape(_SR, 1, NH) + 1e-16)
    return al3.reshape(_SRK, NH)


def _ew_from0(x0s, x0d_exp):
    _, _, feat0 = _edge_geom(x0s, x0d_exp)
    mean0 = jnp.mean(feat0, axis=1, keepdims=True)
    return jax.nn.sigmoid(mean0)


def _stage1_body(h_ref, hs_ref, x0s_ref, xcs_ref, x0d_ref, xcd_ref,
                 wq_ref, wkr_ref, wke_ref, wkh_ref,
                 wvr_ref, wve_ref, wvh_ref, out_ref):
    h = h_ref[...]
    hs = hs_ref[...]
    x0s = x0s_ref[...]
    xcs = xcs_ref[...]
    x0d = x0d_ref[...]
    xcd = xcd_ref[...]

    x0d_exp = _expand(x0d)
    xcd_exp = _expand(xcd)
    rel, dist, feat = _edge_geom(xcs, xcd_exp)
    e_w = _ew_from0(x0s, x0d_exp)
    w = _edge_type_weights(x0s[:, 3:4], x0d_exp[:, 3:4])

    q = _mm(h, wq_ref[...])
    q_exp = _expand(q)
    k_ = _proj(feat, w, wkr_ref, wke_ref[...], hs, wkh_ref[...], HD)
    v_ = _proj(feat, w, wvr_ref, wve_ref[...], hs, wvh_ref[...], HD)

    S = _head_sum_mat()
    sc = _mm(q_exp * k_, S) * (1.0 / math.sqrt(DH))
    al = _soft_over_k(sc)
    al_exp = _mm(al, _head_exp_mat())
    msg = al_exp * v_ * e_w
    agg = jnp.sum(msg.reshape(_SR, K, HD), axis=1)
    z = h + agg
    mean = jnp.mean(z, axis=1, keepdims=True)
    var = jnp.mean((z - mean) ** 2, axis=1, keepdims=True)
    out_ref[...] = (z - mean) / jnp.sqrt(var + 1e-5)


def _stage2_body(h_ref, hs_ref, x0s_ref, xcs_ref, x0d_ref, xcd_ref,
                 uq_ref, ukr_ref, uke_ref, ukh_ref,
                 uvr_ref, uve_ref, uvh_ref, out_ref):
    h = h_ref[...]
    hs = hs_ref[...]
    x0s = x0s_ref[...]
    xcs = xcs_ref[...]
    x0d = x0d_ref[...]
    xcd = xcd_ref[...]

    x0d_exp = _expand(x0d)
    xcd_exp = _expand(xcd)
    rel, dist, feat = _edge_geom(xcs, xcd_exp)
    e_w = _ew_from0(x0s, x0d_exp)
    w = _edge_type_weights(x0s[:, 3:4], x0d_exp[:, 3:4])

    q = _mm(h, uq_ref[...])
    q_exp = _expand(q)
    k_ = _proj(feat, w, ukr_ref, uke_ref[...], hs, ukh_ref[...], HD)
    v_ = _proj(feat, w, uvr_ref, uve_ref[...], hs, uvh_ref[...], NH)

    S = _head_sum_mat()
    sc = _mm(q_exp * k_, S) * (1.0 / math.sqrt(DH))
    al = _soft_over_k(sc)
    coef = jnp.mean(al * v_, axis=1, keepdims=True) * e_w
    dvec = coef * rel / (dist + 1.0)
    delta = jnp.sum(dvec.reshape(_SR, K, 16), axis=1)
    gen = x0d[:, 4:5]
    out_ref[...] = xcd + delta * gen


def _stage_call(body, h, hsrc, x0s, xcs, x0d, xcd, wq, wr, we, wh,
                vr, ve, vh, out_dim):
    nblk = pl.BlockSpec((_SR, HD), lambda i: (i, 0))
    eblk16 = pl.BlockSpec((_SRK, 16), lambda i: (i, 0))
    nblk16 = pl.BlockSpec((_SR, 16), lambda i: (i, 0))

    def full(a):
        return pl.BlockSpec(a.shape, lambda i: tuple(0 for _ in a.shape))

    return pl.pallas_call(
        body,
        grid=(_NSB,),
        in_specs=[
            nblk,
            pl.BlockSpec((_SRK, HD), lambda i: (i, 0)),
            eblk16, eblk16, nblk16, nblk16,
            full(wq), full(wr), full(we), full(wh),
            full(vr), full(ve), full(vh),
        ],
        out_specs=pl.BlockSpec((_SR, out_dim), lambda i: (i, 0)),
        out_shape=jax.ShapeDtypeStruct((NPAD, out_dim), jnp.float32),
        compiler_params=pltpu.CompilerParams(
            dimension_semantics=("arbitrary",)),
    )(h, hsrc, x0s, xcs, x0d, xcd, wq, wr, we, wh, vr, ve, vh)


def _split_w(W):
    dout = W.shape[1]
    Wr = W[: 4 * NG]
    We = jnp.concatenate([W[4 * NG: 4 * NG + 4],
                          jnp.zeros((4, dout), jnp.float32)], axis=0)
    Wh = W[4 * NG + 4:]
    return Wr, We, Wh


def kernel(x, h, batch_idx, lig_flag, gen_flag, Wq, Wk, Wv, Uq, Uk, Uv):
    f32 = jnp.float32
    xp = jnp.zeros((NPAD, 3), f32).at[:N].set(x)
    hp = jnp.zeros((NPAD, HD), f32).at[:N].set(h)
    x2 = jnp.sum(x * x, axis=-1)
    x2p = jnp.zeros((NPAD,), f32).at[:N].set(x2)

    starts = jnp.searchsorted(batch_idx, jnp.arange(NB), side="left")
    ends = jnp.searchsorted(batch_idx, jnp.arange(NB), side="right")
    lo = jnp.zeros((NPAD,), jnp.int32).at[:N].set(
        starts[batch_idx].astype(jnp.int32))
    hi = jnp.zeros((NPAD,), jnp.int32).at[:N].set(
        ends[batch_idx].astype(jnp.int32))

    xp8 = jnp.concatenate([xp, jnp.zeros((NPAD, 5), f32)], axis=1)
    x2row = jnp.broadcast_to(x2p[None, :], (8, NPAD))

    idx = _knn(xp8, xp8, x2p[:, None], x2row,
               lo[:, None], hi[:, None])
    src = idx.reshape(-1)

    ligp = jnp.zeros((NPAD,), f32).at[:N].set(lig_flag.astype(f32))
    genp = jnp.zeros((NPAD,), f32).at[:N].set(gen_flag.astype(f32))

    def node16(xcur):
        return jnp.concatenate(
            [xcur, ligp[:, None], genp[:, None], jnp.zeros((NPAD, 11), f32)],
            axis=1)

    x0tab = node16(xp)
    x0src = _sc_gather(x0tab, src, 512)

    wq_l = [Wq[l] for l in range(L)]
    uq_l = [Uq[l] for l in range(L)]
    wk_l = [_split_w(Wk[l]) for l in range(L)]
    wv_l = [_split_w(Wv[l]) for l in range(L)]
    uk_l = [_split_w(Uk[l]) for l in range(L)]
    uv_l = [_split_w(Uv[l]) for l in range(L)]

    hcur = hp
    xcur_tab = x0tab
    xcur_src = x0src
    for l in range(L):
        hsrc = _sc_gather(hcur, src, 512)
        wr, we, wh = wk_l[l]
        vr, ve, vh = wv_l[l]
        hcur = _stage_call(_stage1_body, hcur, hsrc, x0src, xcur_src,
                           x0tab, xcur_tab, wq_l[l], wr, we, wh, vr, ve, vh,
                           HD)
        hsrc2 = _sc_gather(hcur, src, 512)
        wr, we, wh = uk_l[l]
        vr, ve, vh = uv_l[l]
        xcur_tab = _stage_call(_stage2_body, hcur, hsrc2, x0src, xcur_src,
                               x0tab, xcur_tab, uq_l[l], wr, we, wh,
                               vr, ve, vh, 16)
        if l + 1 < L:
            xcur_src = _sc_gather(xcur_tab, src, 512)

    x_out = xcur_tab[:N, :3]
    h_out = hcur[:N]
    return (x_out, h_out)
